# baseline (speedup 1.0000x reference)
import functools

import jax
import jax.numpy as jnp
from jax import lax
from jax.experimental import pallas as pl
from jax.experimental.pallas import tpu as pltpu

_E = 8
_R = 8
_NEG = -3.0e38


def _pass1_kernel(x_ref, w65_ref, h_ref, xb_ref, s2_ref, *, inv_s):
    s = pl.program_id(1)
    ns = pl.num_programs(1)
    xm = x_ref[0]

    @pl.when(s == 0)
    def _():
        h_ref[...] = jnp.zeros_like(h_ref)

    h_ref[...] += jnp.sum(xm, axis=0)[None, None, :]

    @pl.when(s == ns - 1)
    def _():
        h_ref[...] = h_ref[...] * inv_s

    xb_ref[0] = lax.dot_general(xm, w65_ref[...], (((1,), (0,)), ((), ())),
                                preferred_element_type=jnp.float32)
    x2 = xm * xm
    s2_ref[0] = jnp.sum(x2, axis=1, keepdims=True)


def _pass2_kernel(x_ref, xb_ref, s2_ref, h_ref, gw_ref, gb_ref, a2_ref,
                  b2_ref, gamma_ref, beta_ref, o_ref, delta_ref, wt_ref,
                  st_ref):
    s = pl.program_id(1)

    @pl.when(s == 0)
    def _():
        h = h_ref[0]
        logits = jnp.sum(gw_ref[...] * h, axis=1, keepdims=True) + gb_ref[...]
        iota8 = lax.broadcasted_iota(jnp.int32, (_E, 1), 0)
        m1 = jnp.max(logits)
        i1 = jnp.min(jnp.where(logits == m1, iota8, _E))
        masked = jnp.where(iota8 == i1, _NEG, logits)
        m2 = jnp.max(masked)
        i2 = jnp.min(jnp.where(masked == m2, iota8, _E))
        eb = jnp.exp(m2 - m1)
        denom = 1.0 + eb
        w1 = 1.0 / denom
        w2 = eb / denom
        t = jnp.sum(a2_ref[...] * h, axis=1, keepdims=True)
        e_ids = lax.broadcasted_iota(jnp.int32, (_E * _R, 1), 0) // _R
        wfull = (jnp.where(e_ids == i1, w1, 0.0)
                 + jnp.where(e_ids == i2, w2, 0.0))
        wt = wfull * t * (1.0 / _R)
        delta = jnp.sum(wt * b2_ref[...], axis=0, keepdims=True)
        delta_ref[...] = delta
        wt_ref[...] = wt.reshape(1, _E * _R)
        st_ref[0, 0] = jnp.sum(delta)
        st_ref[0, 1] = jnp.sum(delta * delta)

    D = x_ref.shape[2]
    xm = x_ref[0]
    xb = xb_ref[0]
    s1 = xb[:, 64:65]
    cross = jnp.sum(xb[:, 0:64] * wt_ref[...], axis=1, keepdims=True)
    sum_d = st_ref[0, 0]
    sum_d2 = st_ref[0, 1]
    mu = (s1 + sum_d) * (1.0 / D)
    var = (s2_ref[0] + 2.0 * cross + sum_d2) * (1.0 / D) - mu * mu
    rs = lax.rsqrt(var + 1e-5)
    o_ref[...] = (((xm + delta_ref[...] - mu) * rs) * gamma_ref[...]
                  + beta_ref[...])[None]


def kernel(x, gate_W, gate_b, A_all, B_all, gamma, beta):
    B, S, D = x.shape
    s_blk = 256
    ns = S // s_blk

    A2 = A_all.reshape(_E * _R, D)
    B2 = jnp.transpose(B_all, (0, 2, 1)).reshape(_E * _R, D)
    w65 = jnp.concatenate([B2.T, jnp.ones((D, 1), jnp.float32)], axis=1)
    gb = gate_b.reshape(_E, 1)
    gm = gamma.reshape(1, D)
    bt = beta.reshape(1, D)

    h, xb, s2 = pl.pallas_call(
        functools.partial(_pass1_kernel, inv_s=1.0 / S),
        grid=(B, ns),
        in_specs=[
            pl.BlockSpec((1, s_blk, D), lambda b, s: (b, s, 0)),
            pl.BlockSpec((D, _E * _R + 1), lambda b, s: (0, 0)),
        ],
        out_specs=[
            pl.BlockSpec((1, 1, D), lambda b, s: (b, 0, 0)),
            pl.BlockSpec((1, s_blk, _E * _R + 1), lambda b, s: (b, s, 0)),
            pl.BlockSpec((1, s_blk, 1), lambda b, s: (b, s, 0)),
        ],
        out_shape=[
            jax.ShapeDtypeStruct((B, 1, D), jnp.float32),
            jax.ShapeDtypeStruct((B, S, _E * _R + 1), jnp.float32),
            jax.ShapeDtypeStruct((B, S, 1), jnp.float32),
        ],
    )(x, w65)

    out = pl.pallas_call(
        _pass2_kernel,
        grid=(B, ns),
        in_specs=[
            pl.BlockSpec((1, s_blk, D), lambda b, s: (b, s, 0)),
            pl.BlockSpec((1, s_blk, _E * _R + 1), lambda b, s: (b, s, 0)),
            pl.BlockSpec((1, s_blk, 1), lambda b, s: (b, s, 0)),
            pl.BlockSpec((1, 1, D), lambda b, s: (b, 0, 0)),
            pl.BlockSpec((_E, D), lambda b, s: (0, 0)),
            pl.BlockSpec((_E, 1), lambda b, s: (0, 0)),
            pl.BlockSpec((_E * _R, D), lambda b, s: (0, 0)),
            pl.BlockSpec((_E * _R, D), lambda b, s: (0, 0)),
            pl.BlockSpec((1, D), lambda b, s: (0, 0)),
            pl.BlockSpec((1, D), lambda b, s: (0, 0)),
        ],
        out_specs=pl.BlockSpec((1, s_blk, D), lambda b, s: (b, s, 0)),
        out_shape=jax.ShapeDtypeStruct((B, S, D), jnp.float32),
        scratch_shapes=[
            pltpu.VMEM((1, D), jnp.float32),
            pltpu.VMEM((1, _E * _R), jnp.float32),
            pltpu.SMEM((1, 2), jnp.float32),
        ],
    )(x, xb, s2, h, gate_W, gb, A2, B2, gm, bt)
    return out


# speedup vs baseline: 1.2519x; 1.2519x over previous
import functools

import jax
import jax.numpy as jnp
from jax import lax
from jax.experimental import pallas as pl
from jax.experimental.pallas import tpu as pltpu

_E = 8
_R = 8
_NEG = -3.0e38


def _fused_kernel(x_ref, w65_ref, gw_ref, gb_ref, a2_ref, b2_ref,
                  gamma_ref, beta_ref, o_ref,
                  xc_ref, h_ref, xb_ref, s2_ref, delta_ref, wt_ref, st_ref,
                  *, s_blk, inv_s):
    p = pl.program_id(1)
    s = pl.program_id(2)
    D = x_ref.shape[2]

    @pl.when(p == 0)
    def _phase0():
        xm = x_ref[0]
        xc_ref[pl.ds(s * s_blk, s_blk), :] = xm

        @pl.when(s == 0)
        def _():
            h_ref[...] = jnp.zeros_like(h_ref)

        h_ref[...] += jnp.sum(xm, axis=0)[None, :]

        xb_ref[pl.ds(s * s_blk, s_blk), :] = lax.dot_general(
            xm, w65_ref[...], (((1,), (0,)), ((), ())),
            preferred_element_type=jnp.float32)
        s2_ref[pl.ds(s * s_blk, s_blk), :] = jnp.sum(
            xm * xm, axis=1, keepdims=True)

    @pl.when(p == 1)
    def _phase1():
        @pl.when(s == 0)
        def _():
            h = h_ref[...] * inv_s
            logits = (jnp.sum(gw_ref[...] * h, axis=1, keepdims=True)
                      + gb_ref[...])
            iota8 = lax.broadcasted_iota(jnp.int32, (_E, 1), 0)
            m1 = jnp.max(logits)
            i1 = jnp.min(jnp.where(logits == m1, iota8, _E))
            masked = jnp.where(iota8 == i1, _NEG, logits)
            m2 = jnp.max(masked)
            i2 = jnp.min(jnp.where(masked == m2, iota8, _E))
            eb = jnp.exp(m2 - m1)
            denom = 1.0 + eb
            w1 = 1.0 / denom
            w2 = eb / denom
            t = jnp.sum(a2_ref[...] * h, axis=1, keepdims=True)
            e_ids = lax.broadcasted_iota(jnp.int32, (_E * _R, 1), 0) // _R
            wfull = (jnp.where(e_ids == i1, w1, 0.0)
                     + jnp.where(e_ids == i2, w2, 0.0))
            wt = wfull * t * (1.0 / _R)
            delta = jnp.sum(wt * b2_ref[...], axis=0, keepdims=True)
            delta_ref[...] = delta
            wt_ref[...] = wt.reshape(1, _E * _R)
            st_ref[0, 0] = jnp.sum(delta)
            st_ref[0, 1] = jnp.sum(delta * delta)

        xm = xc_ref[pl.ds(s * s_blk, s_blk), :]
        xb = xb_ref[pl.ds(s * s_blk, s_blk), :]
        s1 = xb[:, 64:65]
        cross = jnp.sum(xb[:, 0:64] * wt_ref[...], axis=1, keepdims=True)
        mu = (s1 + st_ref[0, 0]) * (1.0 / D)
        var = ((s2_ref[pl.ds(s * s_blk, s_blk), :] + 2.0 * cross
                + st_ref[0, 1]) * (1.0 / D) - mu * mu)
        rs = lax.rsqrt(var + 1e-5)
        o_ref[...] = (((xm + delta_ref[...] - mu) * rs) * gamma_ref[...]
                      + beta_ref[...])[None]


def kernel(x, gate_W, gate_b, A_all, B_all, gamma, beta):
    B, S, D = x.shape
    s_blk = 256
    ns = S // s_blk

    A2 = A_all.reshape(_E * _R, D)
    B2 = jnp.transpose(B_all, (0, 2, 1)).reshape(_E * _R, D)
    w65 = jnp.concatenate([B2.T, jnp.ones((D, 1), jnp.float32)], axis=1)
    gb = gate_b.reshape(_E, 1)
    gm = gamma.reshape(1, D)
    bt = beta.reshape(1, D)

    last = ns - 1
    out = pl.pallas_call(
        functools.partial(_fused_kernel, s_blk=s_blk, inv_s=1.0 / S),
        grid=(B, 2, ns),
        in_specs=[
            pl.BlockSpec((1, s_blk, D),
                         lambda b, p, s: (b, s * (1 - p) + last * p, 0)),
            pl.BlockSpec((D, _E * _R + 1), lambda b, p, s: (0, 0)),
            pl.BlockSpec((_E, D), lambda b, p, s: (0, 0)),
            pl.BlockSpec((_E, 1), lambda b, p, s: (0, 0)),
            pl.BlockSpec((_E * _R, D), lambda b, p, s: (0, 0)),
            pl.BlockSpec((_E * _R, D), lambda b, p, s: (0, 0)),
            pl.BlockSpec((1, D), lambda b, p, s: (0, 0)),
            pl.BlockSpec((1, D), lambda b, p, s: (0, 0)),
        ],
        out_specs=pl.BlockSpec((1, s_blk, D), lambda b, p, s: (b, s * p, 0)),
        out_shape=jax.ShapeDtypeStruct((B, S, D), jnp.float32),
        scratch_shapes=[
            pltpu.VMEM((S, D), jnp.float32),
            pltpu.VMEM((1, D), jnp.float32),
            pltpu.VMEM((S, _E * _R + 1), jnp.float32),
            pltpu.VMEM((S, 1), jnp.float32),
            pltpu.VMEM((1, D), jnp.float32),
            pltpu.VMEM((1, _E * _R), jnp.float32),
            pltpu.SMEM((1, 2), jnp.float32),
        ],
    )(x, w65, gate_W, gb, A2, B2, gm, bt)
    return out


# speedup vs baseline: 1.4195x; 1.1339x over previous
import functools

import jax
import jax.numpy as jnp
from jax import lax
from jax.experimental import pallas as pl
from jax.experimental.pallas import tpu as pltpu

_E = 8
_R = 8
_NEG = -3.0e38


def _fused_kernel(x_ref, w65_ref, gw_ref, gb_ref, a2_ref, b2_ref,
                  gamma_ref, beta_ref, o_ref,
                  xc_ref, h_ref, xb_ref, s2_ref, delta_ref, wt_ref, st_ref,
                  *, s_blk, inv_s):
    p = pl.program_id(1)
    s = pl.program_id(2)
    D = x_ref.shape[2]

    @pl.when(p == 0)
    def _phase0():
        xm = x_ref[0]
        xc_ref[pl.ds(s * s_blk, s_blk), :] = xm

        @pl.when(s == 0)
        def _():
            h_ref[...] = jnp.zeros_like(h_ref)

        h_ref[...] += jnp.sum(xm, axis=0)[None, :]

        xb_ref[pl.ds(s * s_blk, s_blk), :] = lax.dot_general(
            xm, w65_ref[...], (((1,), (0,)), ((), ())),
            preferred_element_type=jnp.float32)
        s2_ref[pl.ds(s * s_blk, s_blk), :] = jnp.sum(
            xm * xm, axis=1, keepdims=True)

    @pl.when(p == 1)
    def _phase1():
        @pl.when(s == 0)
        def _():
            h = h_ref[...] * inv_s
            logits = (jnp.sum(gw_ref[...] * h, axis=1, keepdims=True)
                      + gb_ref[...])
            iota8 = lax.broadcasted_iota(jnp.int32, (_E, 1), 0)
            m1 = jnp.max(logits)
            i1 = jnp.min(jnp.where(logits == m1, iota8, _E))
            masked = jnp.where(iota8 == i1, _NEG, logits)
            m2 = jnp.max(masked)
            i2 = jnp.min(jnp.where(masked == m2, iota8, _E))
            eb = jnp.exp(m2 - m1)
            denom = 1.0 + eb
            w1 = 1.0 / denom
            w2 = eb / denom
            t = jnp.sum(a2_ref[...] * h, axis=1, keepdims=True)
            e_ids = lax.broadcasted_iota(jnp.int32, (_E * _R, 1), 0) // _R
            wfull = (jnp.where(e_ids == i1, w1, 0.0)
                     + jnp.where(e_ids == i2, w2, 0.0))
            wt = wfull * t * (1.0 / _R)
            delta = jnp.sum(wt * b2_ref[...], axis=0, keepdims=True)
            delta_ref[...] = delta
            wt_ref[...] = wt.reshape(1, _E * _R)
            st_ref[0, 0] = jnp.sum(delta)
            st_ref[0, 1] = jnp.sum(delta * delta)

        xm = xc_ref[pl.ds(s * s_blk, s_blk), :]
        xb = xb_ref[pl.ds(s * s_blk, s_blk), :]
        s1 = xb[:, 64:65]
        cross = jnp.sum(xb[:, 0:64] * wt_ref[...], axis=1, keepdims=True)
        mu = (s1 + st_ref[0, 0]) * (1.0 / D)
        var = ((s2_ref[pl.ds(s * s_blk, s_blk), :] + 2.0 * cross
                + st_ref[0, 1]) * (1.0 / D) - mu * mu)
        rs = lax.rsqrt(var + 1e-5)
        o_ref[...] = (((xm + delta_ref[...] - mu) * rs) * gamma_ref[...]
                      + beta_ref[...])[None]


def kernel(x, gate_W, gate_b, A_all, B_all, gamma, beta):
    B, S, D = x.shape
    s_blk = 512
    ns = S // s_blk

    A2 = A_all.reshape(_E * _R, D)
    B2 = jnp.transpose(B_all, (0, 2, 1)).reshape(_E * _R, D)
    w65 = jnp.concatenate([B2.T, jnp.ones((D, 1), jnp.float32)], axis=1)
    gb = gate_b.reshape(_E, 1)
    gm = gamma.reshape(1, D)
    bt = beta.reshape(1, D)

    last = ns - 1
    out = pl.pallas_call(
        functools.partial(_fused_kernel, s_blk=s_blk, inv_s=1.0 / S),
        grid=(B, 2, ns),
        in_specs=[
            pl.BlockSpec((1, s_blk, D),
                         lambda b, p, s: (b, s * (1 - p) + last * p, 0)),
            pl.BlockSpec((D, _E * _R + 1), lambda b, p, s: (0, 0)),
            pl.BlockSpec((_E, D), lambda b, p, s: (0, 0)),
            pl.BlockSpec((_E, 1), lambda b, p, s: (0, 0)),
            pl.BlockSpec((_E * _R, D), lambda b, p, s: (0, 0)),
            pl.BlockSpec((_E * _R, D), lambda b, p, s: (0, 0)),
            pl.BlockSpec((1, D), lambda b, p, s: (0, 0)),
            pl.BlockSpec((1, D), lambda b, p, s: (0, 0)),
        ],
        out_specs=pl.BlockSpec((1, s_blk, D), lambda b, p, s: (b, s * p, 0)),
        out_shape=jax.ShapeDtypeStruct((B, S, D), jnp.float32),
        scratch_shapes=[
            pltpu.VMEM((S, D), jnp.float32),
            pltpu.VMEM((1, D), jnp.float32),
            pltpu.VMEM((S, _E * _R + 1), jnp.float32),
            pltpu.VMEM((S, 1), jnp.float32),
            pltpu.VMEM((1, D), jnp.float32),
            pltpu.VMEM((1, _E * _R), jnp.float32),
            pltpu.SMEM((1, 2), jnp.float32),
        ],
    )(x, w65, gate_W, gb, A2, B2, gm, bt)
    return out


# speedup vs baseline: 1.5724x; 1.1077x over previous
import functools

import jax
import jax.numpy as jnp
from jax import lax
from jax.experimental import pallas as pl
from jax.experimental.pallas import tpu as pltpu

_E = 8
_R = 8
_NEG = -3.0e38


def _fused_kernel(x_ref, w65_ref, gw_ref, gb_ref, a2_ref, b2_ref,
                  gamma_ref, beta_ref, o_ref,
                  xc_ref, h_ref, xb_ref, s2_ref, delta_ref, wt_ref, st_ref,
                  sem, *, s_blk, ns, n_b, inv_s):
    b = pl.program_id(0)
    p = pl.program_id(1)
    s = pl.program_id(2)
    D = x_ref.shape[2]
    slot = lax.rem(b, 2)

    def chunk_copy(bb, sl, j):
        return pltpu.make_async_copy(
            x_ref.at[bb, pl.ds(j * s_blk, s_blk), :],
            xc_ref.at[sl, pl.ds(j * s_blk, s_blk), :],
            sem.at[sl, j])

    @pl.when(p == 0)
    def _phase0():
        @pl.when(jnp.logical_and(b == 0, s == 0))
        def _():
            for j in range(ns):
                chunk_copy(0, 0, j).start()

        chunk_copy(b, slot, s).wait()
        xm = xc_ref[slot, pl.ds(s * s_blk, s_blk), :]

        @pl.when(s == 0)
        def _():
            h_ref[...] = jnp.zeros_like(h_ref)

        h_ref[...] += jnp.sum(xm, axis=0)[None, :]

        xb_ref[pl.ds(s * s_blk, s_blk), :] = lax.dot_general(
            xm, w65_ref[...], (((1,), (0,)), ((), ())),
            preferred_element_type=jnp.float32)
        s2_ref[pl.ds(s * s_blk, s_blk), :] = jnp.sum(
            xm * xm, axis=1, keepdims=True)

    @pl.when(p == 1)
    def _phase1():
        @pl.when(s == 0)
        def _():
            @pl.when(b < n_b - 1)
            def _():
                for j in range(ns):
                    chunk_copy(b + 1, 1 - slot, j).start()

            h = h_ref[...] * inv_s
            logits = (jnp.sum(gw_ref[...] * h, axis=1, keepdims=True)
                      + gb_ref[...])
            iota8 = lax.broadcasted_iota(jnp.int32, (_E, 1), 0)
            m1 = jnp.max(logits)
            i1 = jnp.min(jnp.where(logits == m1, iota8, _E))
            masked = jnp.where(iota8 == i1, _NEG, logits)
            m2 = jnp.max(masked)
            i2 = jnp.min(jnp.where(masked == m2, iota8, _E))
            eb = jnp.exp(m2 - m1)
            denom = 1.0 + eb
            w1 = 1.0 / denom
            w2 = eb / denom
            t = jnp.sum(a2_ref[...] * h, axis=1, keepdims=True)
            e_ids = lax.broadcasted_iota(jnp.int32, (_E * _R, 1), 0) // _R
            wfull = (jnp.where(e_ids == i1, w1, 0.0)
                     + jnp.where(e_ids == i2, w2, 0.0))
            wt = wfull * t * (1.0 / _R)
            delta = jnp.sum(wt * b2_ref[...], axis=0, keepdims=True)
            delta_ref[...] = delta
            wt_ref[...] = wt.reshape(1, _E * _R)
            st_ref[0, 0] = jnp.sum(delta)
            st_ref[0, 1] = jnp.sum(delta * delta)

        xm = xc_ref[slot, pl.ds(s * s_blk, s_blk), :]
        xb = xb_ref[pl.ds(s * s_blk, s_blk), :]
        s1 = xb[:, 64:65]
        cross = jnp.sum(xb[:, 0:64] * wt_ref[...], axis=1, keepdims=True)
        mu = (s1 + st_ref[0, 0]) * (1.0 / D)
        var = ((s2_ref[pl.ds(s * s_blk, s_blk), :] + 2.0 * cross
                + st_ref[0, 1]) * (1.0 / D) - mu * mu)
        rs = lax.rsqrt(var + 1e-5)
        o_ref[...] = (((xm + delta_ref[...] - mu) * rs) * gamma_ref[...]
                      + beta_ref[...])[None]


def kernel(x, gate_W, gate_b, A_all, B_all, gamma, beta):
    B, S, D = x.shape
    s_blk = 512
    ns = S // s_blk

    A2 = A_all.reshape(_E * _R, D)
    B2 = jnp.transpose(B_all, (0, 2, 1)).reshape(_E * _R, D)
    w65 = jnp.concatenate([B2.T, jnp.ones((D, 1), jnp.float32)], axis=1)
    gb = gate_b.reshape(_E, 1)
    gm = gamma.reshape(1, D)
    bt = beta.reshape(1, D)

    out = pl.pallas_call(
        functools.partial(_fused_kernel, s_blk=s_blk, ns=ns, n_b=B,
                          inv_s=1.0 / S),
        grid=(B, 2, ns),
        in_specs=[
            pl.BlockSpec(memory_space=pl.ANY),
            pl.BlockSpec((D, _E * _R + 1), lambda b, p, s: (0, 0)),
            pl.BlockSpec((_E, D), lambda b, p, s: (0, 0)),
            pl.BlockSpec((_E, 1), lambda b, p, s: (0, 0)),
            pl.BlockSpec((_E * _R, D), lambda b, p, s: (0, 0)),
            pl.BlockSpec((_E * _R, D), lambda b, p, s: (0, 0)),
            pl.BlockSpec((1, D), lambda b, p, s: (0, 0)),
            pl.BlockSpec((1, D), lambda b, p, s: (0, 0)),
        ],
        out_specs=pl.BlockSpec((1, s_blk, D), lambda b, p, s: (b, s * p, 0)),
        out_shape=jax.ShapeDtypeStruct((B, S, D), jnp.float32),
        scratch_shapes=[
            pltpu.VMEM((2, S, D), jnp.float32),
            pltpu.VMEM((1, D), jnp.float32),
            pltpu.VMEM((S, _E * _R + 1), jnp.float32),
            pltpu.VMEM((S, 1), jnp.float32),
            pltpu.VMEM((1, D), jnp.float32),
            pltpu.VMEM((1, _E * _R), jnp.float32),
            pltpu.SMEM((1, 2), jnp.float32),
            pltpu.SemaphoreType.DMA((2, S // s_blk)),
        ],
    )(x, w65, gate_W, gb, A2, B2, gm, bt)
    return out


# speedup vs baseline: 1.8727x; 1.1910x over previous
import functools

import jax
import jax.numpy as jnp
from jax import lax
from jax.experimental import pallas as pl
from jax.experimental.pallas import tpu as pltpu

_E = 8
_R = 8
_NEG = -3.0e38


def _fused_kernel(x_ref, w65_ref, gw_ref, gb_ref, a2_ref, b2_ref,
                  gamma_ref, beta_ref, o_ref,
                  xc_ref, h_ref, xb_ref, s2_ref, delta_ref, wt_ref, st_ref,
                  sem, *, s_blk, ns, n_b, inv_s):
    b = pl.program_id(0)
    s = pl.program_id(1)
    D = x_ref.shape[2]
    slot = lax.rem(b, 2)
    pslot = 1 - slot

    def chunk_copy(bb, sl, j):
        return pltpu.make_async_copy(
            x_ref.at[bb, pl.ds(j * s_blk, s_blk), :],
            xc_ref.at[sl, pl.ds(j * s_blk, s_blk), :],
            sem.at[sl, j])

    @pl.when(jnp.logical_and(b == 0, s == 0))
    def _():
        for j in range(ns):
            chunk_copy(0, 0, j).start()

    @pl.when(jnp.logical_and(s == 0, jnp.logical_and(b >= 1, b < n_b)))
    def _():
        chunk_copy(b, slot, ns - 1).start()

    @pl.when(jnp.logical_and(s >= 1, b + 1 < n_b))
    def _():
        chunk_copy(b + 1, pslot, s - 1).start()

    @pl.when(b < n_b)
    def _stats():
        chunk_copy(b, slot, s).wait()
        xm = xc_ref[slot, pl.ds(s * s_blk, s_blk), :]

        @pl.when(s == 0)
        def _():
            h_ref[slot] = jnp.zeros((1, D), jnp.float32)

        h_ref[slot] += jnp.sum(xm, axis=0)[None, :]

        xb_ref[slot, pl.ds(s * s_blk, s_blk), :] = lax.dot_general(
            xm, w65_ref[...], (((1,), (0,)), ((), ())),
            preferred_element_type=jnp.float32)
        s2_ref[slot, pl.ds(s * s_blk, s_blk), :] = jnp.sum(
            xm * xm, axis=1, keepdims=True)

    @pl.when(b >= 1)
    def _normalize():
        @pl.when(s == 0)
        def _():
            h = h_ref[pslot] * inv_s
            logits = (jnp.sum(gw_ref[...] * h, axis=1, keepdims=True)
                      + gb_ref[...])
            iota8 = lax.broadcasted_iota(jnp.int32, (_E, 1), 0)
            m1 = jnp.max(logits)
            i1 = jnp.min(jnp.where(logits == m1, iota8, _E))
            masked = jnp.where(iota8 == i1, _NEG, logits)
            m2 = jnp.max(masked)
            i2 = jnp.min(jnp.where(masked == m2, iota8, _E))
            eb = jnp.exp(m2 - m1)
            denom = 1.0 + eb
            w1 = 1.0 / denom
            w2 = eb / denom
            t = jnp.sum(a2_ref[...] * h, axis=1, keepdims=True)
            e_ids = lax.broadcasted_iota(jnp.int32, (_E * _R, 1), 0) // _R
            wfull = (jnp.where(e_ids == i1, w1, 0.0)
                     + jnp.where(e_ids == i2, w2, 0.0))
            wt = wfull * t * (1.0 / _R)
            delta = jnp.sum(wt * b2_ref[...], axis=0, keepdims=True)
            delta_ref[...] = delta
            wt_ref[...] = wt.reshape(1, _E * _R)
            st_ref[0, 0] = jnp.sum(delta)
            st_ref[0, 1] = jnp.sum(delta * delta)

        xm = xc_ref[pslot, pl.ds(s * s_blk, s_blk), :]
        xb = xb_ref[pslot, pl.ds(s * s_blk, s_blk), :]
        s1 = xb[:, 64:65]
        cross = jnp.sum(xb[:, 0:64] * wt_ref[...], axis=1, keepdims=True)
        mu = (s1 + st_ref[0, 0]) * (1.0 / D)
        var = ((s2_ref[pslot, pl.ds(s * s_blk, s_blk), :] + 2.0 * cross
                + st_ref[0, 1]) * (1.0 / D) - mu * mu)
        rs = lax.rsqrt(var + 1e-5)
        o_ref[...] = (((xm + delta_ref[...] - mu) * rs) * gamma_ref[...]
                      + beta_ref[...])[None]


def kernel(x, gate_W, gate_b, A_all, B_all, gamma, beta):
    B, S, D = x.shape
    s_blk = 512
    ns = S // s_blk

    A2 = A_all.reshape(_E * _R, D)
    B2 = jnp.transpose(B_all, (0, 2, 1)).reshape(_E * _R, D)
    w65 = jnp.concatenate([B2.T, jnp.ones((D, 1), jnp.float32)], axis=1)
    gb = gate_b.reshape(_E, 1)
    gm = gamma.reshape(1, D)
    bt = beta.reshape(1, D)

    out = pl.pallas_call(
        functools.partial(_fused_kernel, s_blk=s_blk, ns=ns, n_b=B,
                          inv_s=1.0 / S),
        grid=(B + 1, ns),
        in_specs=[
            pl.BlockSpec(memory_space=pl.ANY),
            pl.BlockSpec((D, _E * _R + 1), lambda b, s: (0, 0)),
            pl.BlockSpec((_E, D), lambda b, s: (0, 0)),
            pl.BlockSpec((_E, 1), lambda b, s: (0, 0)),
            pl.BlockSpec((_E * _R, D), lambda b, s: (0, 0)),
            pl.BlockSpec((_E * _R, D), lambda b, s: (0, 0)),
            pl.BlockSpec((1, D), lambda b, s: (0, 0)),
            pl.BlockSpec((1, D), lambda b, s: (0, 0)),
        ],
        out_specs=pl.BlockSpec(
            (1, s_blk, D),
            lambda b, s: (jnp.maximum(b - 1, 0), s * jnp.minimum(b, 1), 0)),
        out_shape=jax.ShapeDtypeStruct((B, S, D), jnp.float32),
        scratch_shapes=[
            pltpu.VMEM((2, S, D), jnp.float32),
            pltpu.VMEM((2, 1, D), jnp.float32),
            pltpu.VMEM((2, S, _E * _R + 1), jnp.float32),
            pltpu.VMEM((2, S, 1), jnp.float32),
            pltpu.VMEM((1, D), jnp.float32),
            pltpu.VMEM((1, _E * _R), jnp.float32),
            pltpu.SMEM((1, 2), jnp.float32),
            pltpu.SemaphoreType.DMA((2, S // s_blk)),
        ],
    )(x, w65, gate_W, gb, A2, B2, gm, bt)
    return out
